# Initial kernel scaffold; baseline (speedup 1.0000x reference)
import jax
import jax.numpy as jnp
from jax.experimental import pallas as pl


def kernel(adj_edge_index, adj_values, features, W, b):
    raise NotImplementedError("write your pallas kernel here")


# speedup vs baseline: 6.4457x; 6.4457x over previous
import functools

import jax
import jax.numpy as jnp
from jax import lax
from jax.experimental import pallas as pl
from jax.experimental.pallas import tpu as pltpu
from jax.experimental.pallas import tpu_sc as plsc

N = 10000
E = 320000
D = 128
NC = 2
NS = 16
EPW = E // (NC * NS)
K = 80
CPS = 25
SB = EPW // (K * CPS)
ROWS_A = 624
ROWS_LAST = N - 15 * ROWS_A
RB = N // 10


def _linear_body(f_ref, wt_ref, b_ref, o_ref):
    o_ref[...] = (
        jnp.dot(f_ref[...], wt_ref[...], preferred_element_type=jnp.float32)
        + b_ref[...]
    )


def _linear(features, Wt, b2):
    return pl.pallas_call(
        _linear_body,
        grid=(10,),
        in_specs=[
            pl.BlockSpec((RB, D), lambda i: (i, 0)),
            pl.BlockSpec((D, D), lambda i: (0, 0)),
            pl.BlockSpec((1, D), lambda i: (0, 0)),
        ],
        out_specs=pl.BlockSpec((RB, D), lambda i: (i, 0)),
        out_shape=jax.ShapeDtypeStruct((N, D), jnp.float32),
    )(features, Wt, b2)


def _combine_body(a_ref, b_ref, o_ref):
    o_ref[...] = a_ref[0] + b_ref[0]


def _combine(out2):
    return pl.pallas_call(
        _combine_body,
        grid=(10,),
        in_specs=[
            pl.BlockSpec((1, RB, D), lambda i: (0, i, 0)),
            pl.BlockSpec((1, RB, D), lambda i: (1, i, 0)),
        ],
        out_specs=pl.BlockSpec((RB, D), lambda i: (i, 0)),
        out_shape=jax.ShapeDtypeStruct((N, D), jnp.float32),
    )(out2, out2)


def _sc_body(src_hbm, dst_hbm, val_hbm, h_hbm, out_hbm,
             src_v, dst_v, val_v, rows_v, acc_sh, sem):
    c = lax.axis_index("c")
    s = lax.axis_index("s")

    zero16 = jnp.zeros((16,), jnp.float32)

    def zrow(r, carry):
        for g in range(D // 16):
            rows_v[r, pl.ds(g * 16, 16)] = zero16
        return carry

    lax.fori_loop(0, K, zrow, 0)

    base = s * ROWS_A
    nfull = ROWS_A // K

    def zacc(k, carry):
        pltpu.sync_copy(rows_v, acc_sh.at[pl.ds(base + k * K, K)])
        return carry

    lax.fori_loop(0, nfull, zacc, 0)

    @pl.when(s < 15)
    def _():
        pltpu.sync_copy(rows_v.at[pl.ds(0, ROWS_A - nfull * K)],
                        acc_sh.at[pl.ds(base + nfull * K, ROWS_A - nfull * K)])

    @pl.when(s == 15)
    def _():
        def ztail(k, carry):
            pltpu.sync_copy(rows_v,
                            acc_sh.at[pl.ds(base + nfull * K + k * K, K)])
            return carry
        lax.fori_loop(0, (ROWS_LAST - nfull * K) // K, ztail, 0)

    plsc.subcore_barrier()

    def chunk(j, carry):
        pltpu.async_copy(h_hbm.at[src_v.at[j]], rows_v, sem).wait()
        for g in range(K // 16):
            vv = val_v[j, pl.ds(g * 16, 16)]
            for e in range(16):
                sp = vv.at[jnp.full((16,), e, jnp.int32)].get(
                    mode="promise_in_bounds")
                r = g * 16 + e
                for q in range(D // 16):
                    sl = pl.ds(q * 16, 16)
                    rows_v[r, sl] = rows_v[r, sl] * sp
        pltpu.sync_copy(rows_v, acc_sh.at[dst_v.at[j]], add=True)
        return carry

    def superchunk(t, carry):
        pltpu.sync_copy(src_hbm.at[c, s, t], src_v)
        pltpu.sync_copy(dst_hbm.at[c, s, t], dst_v)
        pltpu.sync_copy(val_hbm.at[c, s, t], val_v)
        lax.fori_loop(0, CPS, chunk, 0)
        return carry

    lax.fori_loop(0, SB, superchunk, 0)

    plsc.subcore_barrier()

    @pl.when(s < 15)
    def _():
        pltpu.sync_copy(acc_sh.at[pl.ds(base, ROWS_A)],
                        out_hbm.at[c, pl.ds(base, ROWS_A)])

    @pl.when(s == 15)
    def _():
        pltpu.sync_copy(acc_sh.at[pl.ds(base, ROWS_LAST)],
                        out_hbm.at[c, pl.ds(base, ROWS_LAST)])


_sc_kernel = functools.partial(
    pl.kernel,
    out_type=jax.ShapeDtypeStruct((NC, N, D), jnp.float32),
    mesh=plsc.VectorSubcoreMesh(
        core_axis_name="c", subcore_axis_name="s",
        num_cores=NC, num_subcores=NS),
    scratch_types=[
        pltpu.VMEM((CPS, K), jnp.int32),
        pltpu.VMEM((CPS, K), jnp.int32),
        pltpu.VMEM((CPS, K), jnp.float32),
        pltpu.VMEM((K, D), jnp.float32),
        pltpu.VMEM_SHARED((N, D), jnp.float32),
        pltpu.SemaphoreType.DMA,
    ],
)(_sc_body)


@jax.jit
def kernel(adj_edge_index, adj_values, features, W, b):
    src = adj_edge_index[0].astype(jnp.int32).reshape(NC, NS, SB, CPS, K)
    dst = adj_edge_index[1].astype(jnp.int32).reshape(NC, NS, SB, CPS, K)
    val = adj_values.reshape(NC, NS, SB, CPS, K)
    h = _linear(features, W.T, b.reshape(1, D))
    out2 = _sc_kernel(src, dst, val, h)
    return _combine(out2)


# speedup vs baseline: 8.1844x; 1.2697x over previous
import functools

import jax
import jax.numpy as jnp
from jax import lax
from jax.experimental import pallas as pl
from jax.experimental.pallas import tpu as pltpu
from jax.experimental.pallas import tpu_sc as plsc

N = 10000
E = 320000
D = 128
NC = 2
NS = 16
EPW = E // (NC * NS)
K = 80
CPS = 25
SB = EPW // (K * CPS)
NCHUNK = SB * CPS
ROWS_A = 624
ROWS_LAST = N - 15 * ROWS_A
RB = N // 10


def _linear_body(f_ref, wt_ref, b_ref, o_ref):
    o_ref[...] = (
        jnp.dot(f_ref[...], wt_ref[...], preferred_element_type=jnp.float32)
        + b_ref[...]
    )


def _linear(features, Wt, b2):
    return pl.pallas_call(
        _linear_body,
        grid=(10,),
        in_specs=[
            pl.BlockSpec((RB, D), lambda i: (i, 0)),
            pl.BlockSpec((D, D), lambda i: (0, 0)),
            pl.BlockSpec((1, D), lambda i: (0, 0)),
        ],
        out_specs=pl.BlockSpec((RB, D), lambda i: (i, 0)),
        out_shape=jax.ShapeDtypeStruct((N, D), jnp.float32),
    )(features, Wt, b2)


def _combine_body(a_ref, b_ref, o_ref):
    o_ref[...] = a_ref[0] + b_ref[0]


def _combine(out2):
    return pl.pallas_call(
        _combine_body,
        grid=(10,),
        in_specs=[
            pl.BlockSpec((1, RB, D), lambda i: (0, i, 0)),
            pl.BlockSpec((1, RB, D), lambda i: (1, i, 0)),
        ],
        out_specs=pl.BlockSpec((RB, D), lambda i: (i, 0)),
        out_shape=jax.ShapeDtypeStruct((N, D), jnp.float32),
    )(out2, out2)


def _sc_body(src_hbm, dst_hbm, val_hbm, h_hbm, out_hbm,
             src_v, dst_v, val_v, rows0_v, rows1_v, rows2_v, acc_sh,
             sg0, sg1, sg2, ss0, ss1, ss2):
    c = lax.axis_index("c")
    s = lax.axis_index("s")
    rows_v = rows0_v

    zero16 = jnp.zeros((16,), jnp.float32)

    def zrow(r, carry):
        for g in range(D // 16):
            rows_v[r, pl.ds(g * 16, 16)] = zero16
        return carry

    lax.fori_loop(0, K, zrow, 0)

    base = s * ROWS_A
    nfull = ROWS_A // K

    def zacc(k, carry):
        pltpu.sync_copy(rows_v, acc_sh.at[pl.ds(base + k * K, K)])
        return carry

    lax.fori_loop(0, nfull, zacc, 0)

    @pl.when(s < 15)
    def _():
        pltpu.sync_copy(rows_v.at[pl.ds(0, ROWS_A - nfull * K)],
                        acc_sh.at[pl.ds(base + nfull * K, ROWS_A - nfull * K)])

    @pl.when(s == 15)
    def _():
        def ztail(k, carry):
            pltpu.sync_copy(rows_v,
                            acc_sh.at[pl.ds(base + nfull * K + k * K, K)])
            return carry
        lax.fori_loop(0, (ROWS_LAST - nfull * K) // K, ztail, 0)

    plsc.subcore_barrier()

    def stage(u):
        pltpu.sync_copy(src_hbm.at[c, s, u], src_v)
        pltpu.sync_copy(dst_hbm.at[c, s, u], dst_v.at[lax.rem(u, 2)])

    def stage_val(u):
        pltpu.sync_copy(val_hbm.at[c, s, u], val_v)

    def scale(buf, r):
        for g in range(K // 16):
            vv = val_v[r, pl.ds(g * 16, 16)]
            for e in range(16):
                sp = vv.at[jnp.full((16,), e, jnp.int32)].get(
                    mode="promise_in_bounds")
                row = g * 16 + e
                for q in range(D // 16):
                    slc = pl.ds(q * 16, 16)
                    buf[row, slc] = buf[row, slc] * sp

    def step(t, bufA, sgA, ssA, bufB, sgB, ssB):
        t = jnp.int32(t)
        sl = lax.rem(t // CPS, 2)
        r = lax.rem(t, CPS)
        pltpu.make_async_copy(h_hbm.at[src_v.at[r]], bufA, sgA).wait()

        @pl.when(t >= 2)
        def _():
            pltpu.make_async_copy(
                bufB, acc_sh.at[dst_v.at[sl, r]], ssB).wait()

        @pl.when(jnp.logical_and(lax.rem(t + 1, CPS) == 0,
                                 t + 1 < NCHUNK))
        def _():
            stage((t + 1) // CPS)

        @pl.when(t + 1 < NCHUNK)
        def _():
            pltpu.async_copy(
                h_hbm.at[src_v.at[lax.rem(t + 1, CPS)]], bufB, sgB)

        scale(bufA, r)
        pltpu.async_copy(bufA, acc_sh.at[dst_v.at[sl, r]], ssA, add=True)

        @pl.when(jnp.logical_and(lax.rem(t + 1, CPS) == 0,
                                 t + 1 < NCHUNK))
        def _():
            stage_val((t + 1) // CPS)

    stage(0)
    stage_val(0)
    pltpu.async_copy(h_hbm.at[src_v.at[0]], rows0_v, sg0)

    def triple(p, carry):
        t = 3 * p
        step(t, rows0_v, sg0, ss0, rows1_v, sg1, ss1)
        step(t + 1, rows1_v, sg1, ss1, rows2_v, sg2, ss2)
        step(t + 2, rows2_v, sg2, ss2, rows0_v, sg0, ss0)
        return carry

    ntr = (NCHUNK - 2) // 3
    lax.fori_loop(0, ntr, triple, 0)
    step(NCHUNK - 2, rows0_v, sg0, ss0, rows1_v, sg1, ss1)
    step(NCHUNK - 1, rows1_v, sg1, ss1, rows2_v, sg2, ss2)
    lastsl = lax.rem((NCHUNK - 1) // CPS, 2)
    pltpu.make_async_copy(rows0_v, acc_sh.at[dst_v.at[lastsl, 0]], ss0).wait()
    pltpu.make_async_copy(rows1_v, acc_sh.at[dst_v.at[lastsl, 0]], ss1).wait()

    plsc.subcore_barrier()

    @pl.when(s < 15)
    def _():
        pltpu.sync_copy(acc_sh.at[pl.ds(base, ROWS_A)],
                        out_hbm.at[c, pl.ds(base, ROWS_A)])

    @pl.when(s == 15)
    def _():
        pltpu.sync_copy(acc_sh.at[pl.ds(base, ROWS_LAST)],
                        out_hbm.at[c, pl.ds(base, ROWS_LAST)])


_sc_kernel = functools.partial(
    pl.kernel,
    out_type=jax.ShapeDtypeStruct((NC, N, D), jnp.float32),
    mesh=plsc.VectorSubcoreMesh(
        core_axis_name="c", subcore_axis_name="s",
        num_cores=NC, num_subcores=NS),
    scratch_types=[
        pltpu.VMEM((CPS, K), jnp.int32),
        pltpu.VMEM((2, CPS, K), jnp.int32),
        pltpu.VMEM((CPS, K), jnp.float32),
        pltpu.VMEM((K, D), jnp.float32),
        pltpu.VMEM((K, D), jnp.float32),
        pltpu.VMEM((K, D), jnp.float32),
        pltpu.VMEM_SHARED((N, D), jnp.float32),
        pltpu.SemaphoreType.DMA,
        pltpu.SemaphoreType.DMA,
        pltpu.SemaphoreType.DMA,
        pltpu.SemaphoreType.DMA,
        pltpu.SemaphoreType.DMA,
        pltpu.SemaphoreType.DMA,
    ],
)(_sc_body)


@jax.jit
def kernel(adj_edge_index, adj_values, features, W, b):
    src = adj_edge_index[0].astype(jnp.int32).reshape(NC, NS, SB, CPS, K)
    dst = adj_edge_index[1].astype(jnp.int32).reshape(NC, NS, SB, CPS, K)
    val = adj_values.reshape(NC, NS, SB, CPS, K)
    h = _linear(features, W.T, b.reshape(1, D))
    out2 = _sc_kernel(src, dst, val, h)
    return _combine(out2)


# speedup vs baseline: 8.4327x; 1.0303x over previous
import functools

import jax
import jax.numpy as jnp
from jax import lax
from jax.experimental import pallas as pl
from jax.experimental.pallas import tpu as pltpu
from jax.experimental.pallas import tpu_sc as plsc

N = 10000
E = 320000
D = 128
NC = 2
NS = 16
EPW = E // (NC * NS)
K = 80
CPS = 25
SB = EPW // (K * CPS)
NCHUNK = SB * CPS
ROWS_A = 624
ROWS_LAST = N - 15 * ROWS_A
RB = N // 10


def _linear_body(f_ref, wt_ref, b_ref, o_ref):
    o_ref[...] = (
        jnp.dot(f_ref[...], wt_ref[...], preferred_element_type=jnp.float32)
        + b_ref[...]
    )


def _linear(features, Wt, b2):
    return pl.pallas_call(
        _linear_body,
        grid=(10,),
        in_specs=[
            pl.BlockSpec((RB, D), lambda i: (i, 0)),
            pl.BlockSpec((D, D), lambda i: (0, 0)),
            pl.BlockSpec((1, D), lambda i: (0, 0)),
        ],
        out_specs=pl.BlockSpec((RB, D), lambda i: (i, 0)),
        out_shape=jax.ShapeDtypeStruct((N, D), jnp.float32),
    )(features, Wt, b2)


def _combine_body(a_ref, b_ref, o_ref):
    o_ref[...] = a_ref[0] + b_ref[0]


def _combine(out2):
    return pl.pallas_call(
        _combine_body,
        grid=(10,),
        in_specs=[
            pl.BlockSpec((1, RB, D), lambda i: (0, i, 0)),
            pl.BlockSpec((1, RB, D), lambda i: (1, i, 0)),
        ],
        out_specs=pl.BlockSpec((RB, D), lambda i: (i, 0)),
        out_shape=jax.ShapeDtypeStruct((N, D), jnp.float32),
    )(out2, out2)


def _sc_body(src_hbm, dst_hbm, val_hbm, h_hbm, out_hbm,
             src_v, dst_v, val_v, rows0_v, rows1_v, rows2_v, acc_sh,
             sg0, sg1, sg2, ss0, ss1, ss2):
    c = lax.axis_index("c")
    s = lax.axis_index("s")
    rows_v = rows0_v

    zero16 = jnp.zeros((16,), jnp.float32)

    def zrow(r, carry):
        for g in range(D // 16):
            rows_v[r, pl.ds(g * 16, 16)] = zero16
        return carry

    lax.fori_loop(0, K, zrow, 0)

    base = s * ROWS_A
    nfull = ROWS_A // K

    def zacc(k, carry):
        pltpu.sync_copy(rows_v, acc_sh.at[pl.ds(base + k * K, K)])
        return carry

    lax.fori_loop(0, nfull, zacc, 0)

    @pl.when(s < 15)
    def _():
        pltpu.sync_copy(rows_v.at[pl.ds(0, ROWS_A - nfull * K)],
                        acc_sh.at[pl.ds(base + nfull * K, ROWS_A - nfull * K)])

    @pl.when(s == 15)
    def _():
        def ztail(k, carry):
            pltpu.sync_copy(rows_v,
                            acc_sh.at[pl.ds(base + nfull * K + k * K, K)])
            return carry
        lax.fori_loop(0, (ROWS_LAST - nfull * K) // K, ztail, 0)

    plsc.subcore_barrier()

    def stage(u):
        pltpu.sync_copy(src_hbm.at[c, s, u], src_v)
        pltpu.sync_copy(dst_hbm.at[c, s, u], dst_v.at[lax.rem(u, 2)])

    def stage_val(u):
        pltpu.sync_copy(val_hbm.at[c, s, u], val_v)

    def scale(buf, r):
        for g in range(K // 16):
            vv = val_v[r, pl.ds(g * 16, 16)]
            for e in range(16):
                sp = vv.at[jnp.full((16,), e, jnp.int32)].get(
                    mode="promise_in_bounds")
                row = g * 16 + e
                for q in range(D // 16):
                    slc = pl.ds(q * 16, 16)
                    buf[row, slc] = buf[row, slc] * sp

    def step(t, bufA, sgA, ssA, bufB, sgB, ssB):
        t = jnp.int32(t)
        sl = lax.rem(t // CPS, 2)
        r = lax.rem(t, CPS)

        boundary = jnp.logical_and(lax.rem(t + 1, CPS) == 0,
                                   t + 1 < NCHUNK)

        @pl.when(t >= 2)
        def _():
            pltpu.make_async_copy(
                bufB, acc_sh.at[dst_v.at[sl, r]], ssB).wait()

        @pl.when(boundary)
        def _():
            pltpu.make_async_copy(h_hbm.at[src_v.at[r]], bufA, sgA).wait()
            stage((t + 1) // CPS)

        @pl.when(t + 1 < NCHUNK)
        def _():
            pltpu.async_copy(
                h_hbm.at[src_v.at[lax.rem(t + 1, CPS)]], bufB, sgB)

        @pl.when(jnp.logical_not(boundary))
        def _():
            pltpu.make_async_copy(h_hbm.at[src_v.at[r]], bufA, sgA).wait()

        scale(bufA, r)
        pltpu.async_copy(bufA, acc_sh.at[dst_v.at[sl, r]], ssA, add=True)

        @pl.when(boundary)
        def _():
            stage_val((t + 1) // CPS)

    stage(0)
    stage_val(0)
    pltpu.async_copy(h_hbm.at[src_v.at[0]], rows0_v, sg0)

    def triple(p, carry):
        t = 3 * p
        step(t, rows0_v, sg0, ss0, rows1_v, sg1, ss1)
        step(t + 1, rows1_v, sg1, ss1, rows2_v, sg2, ss2)
        step(t + 2, rows2_v, sg2, ss2, rows0_v, sg0, ss0)
        return carry

    ntr = (NCHUNK - 2) // 3
    lax.fori_loop(0, ntr, triple, 0)
    step(NCHUNK - 2, rows0_v, sg0, ss0, rows1_v, sg1, ss1)
    step(NCHUNK - 1, rows1_v, sg1, ss1, rows2_v, sg2, ss2)
    lastsl = lax.rem((NCHUNK - 1) // CPS, 2)
    pltpu.make_async_copy(rows0_v, acc_sh.at[dst_v.at[lastsl, 0]], ss0).wait()
    pltpu.make_async_copy(rows1_v, acc_sh.at[dst_v.at[lastsl, 0]], ss1).wait()

    plsc.subcore_barrier()

    @pl.when(s < 15)
    def _():
        pltpu.sync_copy(acc_sh.at[pl.ds(base, ROWS_A)],
                        out_hbm.at[c, pl.ds(base, ROWS_A)])

    @pl.when(s == 15)
    def _():
        pltpu.sync_copy(acc_sh.at[pl.ds(base, ROWS_LAST)],
                        out_hbm.at[c, pl.ds(base, ROWS_LAST)])


_sc_kernel = functools.partial(
    pl.kernel,
    out_type=jax.ShapeDtypeStruct((NC, N, D), jnp.float32),
    mesh=plsc.VectorSubcoreMesh(
        core_axis_name="c", subcore_axis_name="s",
        num_cores=NC, num_subcores=NS),
    scratch_types=[
        pltpu.VMEM((CPS, K), jnp.int32),
        pltpu.VMEM((2, CPS, K), jnp.int32),
        pltpu.VMEM((CPS, K), jnp.float32),
        pltpu.VMEM((K, D), jnp.float32),
        pltpu.VMEM((K, D), jnp.float32),
        pltpu.VMEM((K, D), jnp.float32),
        pltpu.VMEM_SHARED((N, D), jnp.float32),
        pltpu.SemaphoreType.DMA,
        pltpu.SemaphoreType.DMA,
        pltpu.SemaphoreType.DMA,
        pltpu.SemaphoreType.DMA,
        pltpu.SemaphoreType.DMA,
        pltpu.SemaphoreType.DMA,
    ],
)(_sc_body)


@jax.jit
def kernel(adj_edge_index, adj_values, features, W, b):
    src = adj_edge_index[0].astype(jnp.int32).reshape(NC, NS, SB, CPS, K)
    dst = adj_edge_index[1].astype(jnp.int32).reshape(NC, NS, SB, CPS, K)
    val = adj_values.reshape(NC, NS, SB, CPS, K)
    h = _linear(features, W.T, b.reshape(1, D))
    out2 = _sc_kernel(src, dst, val, h)
    return _combine(out2)


# speedup vs baseline: 8.5308x; 1.0116x over previous
import functools

import jax
import jax.numpy as jnp
from jax import lax
from jax.experimental import pallas as pl
from jax.experimental.pallas import tpu as pltpu
from jax.experimental.pallas import tpu_sc as plsc

N = 10000
E = 320000
D = 128
NC = 2
NS = 16
EPW = E // (NC * NS)
K = 80
CPS = 25
SB = EPW // (K * CPS)
NCHUNK = SB * CPS
ROWS_A = 624
ROWS_LAST = N - 15 * ROWS_A
RB = N // 10


def _linear_body(f_ref, wt_ref, b_ref, o_ref):
    o_ref[...] = (
        jnp.dot(f_ref[...], wt_ref[...], preferred_element_type=jnp.float32)
        + b_ref[...]
    )


def _linear(features, Wt, b2):
    return pl.pallas_call(
        _linear_body,
        grid=(10,),
        in_specs=[
            pl.BlockSpec((RB, D), lambda i: (i, 0)),
            pl.BlockSpec((D, D), lambda i: (0, 0)),
            pl.BlockSpec((1, D), lambda i: (0, 0)),
        ],
        out_specs=pl.BlockSpec((RB, D), lambda i: (i, 0)),
        out_shape=jax.ShapeDtypeStruct((N, D), jnp.float32),
    )(features, Wt, b2)


def _combine_body(a_ref, b_ref, o_ref):
    o_ref[...] = a_ref[0] + b_ref[0]


def _combine(out2):
    return pl.pallas_call(
        _combine_body,
        grid=(10,),
        in_specs=[
            pl.BlockSpec((1, RB, D), lambda i: (0, i, 0)),
            pl.BlockSpec((1, RB, D), lambda i: (1, i, 0)),
        ],
        out_specs=pl.BlockSpec((RB, D), lambda i: (i, 0)),
        out_shape=jax.ShapeDtypeStruct((N, D), jnp.float32),
    )(out2, out2)


def _sc_body(src_hbm, dst_hbm, val_hbm, h_hbm, out_hbm,
             src_v, dst_v, val_v, rows0_v, rows1_v, rows2_v, acc_sh,
             sg0, sg1, sg2, ss0, ss1, ss2):
    c = lax.axis_index("c")
    s = lax.axis_index("s")
    rows_v = rows0_v

    zero16 = jnp.zeros((16,), jnp.float32)

    def zrow(r, carry):
        for g in range(D // 16):
            rows_v[r, pl.ds(g * 16, 16)] = zero16
        return carry

    lax.fori_loop(0, K, zrow, 0)

    base = s * ROWS_A
    nfull = ROWS_A // K

    rem = ROWS_A - nfull * K
    ntail = (ROWS_LAST - nfull * K) // K

    def zacc(k, carry):
        pltpu.async_copy(rows_v, acc_sh.at[pl.ds(base + k * K, K)], sg1)
        return carry

    lax.fori_loop(0, nfull, zacc, 0)

    @pl.when(s < 15)
    def _():
        pltpu.async_copy(rows_v.at[pl.ds(0, rem)],
                         acc_sh.at[pl.ds(base + nfull * K, rem)], sg1)

    @pl.when(s == 15)
    def _():
        def ztail(k, carry):
            pltpu.async_copy(rows_v,
                             acc_sh.at[pl.ds(base + nfull * K + k * K, K)],
                             sg1)
            return carry
        lax.fori_loop(0, ntail, ztail, 0)

    def stage(u):
        pltpu.sync_copy(src_hbm.at[c, s, u], src_v)
        pltpu.sync_copy(dst_hbm.at[c, s, u], dst_v.at[lax.rem(u, 2)])

    def stage_val(u):
        pltpu.sync_copy(val_hbm.at[c, s, u], val_v)

    def scale(buf, r):
        for g in range(K // 16):
            vv = val_v[r, pl.ds(g * 16, 16)]
            for e in range(16):
                sp = vv.at[jnp.full((16,), e, jnp.int32)].get(
                    mode="promise_in_bounds")
                row = g * 16 + e
                for q in range(D // 16):
                    slc = pl.ds(q * 16, 16)
                    buf[row, slc] = buf[row, slc] * sp

    def step(t, bufA, sgA, ssA, bufB, sgB, ssB):
        t = jnp.int32(t)
        sl = lax.rem(t // CPS, 2)
        r = lax.rem(t, CPS)

        boundary = jnp.logical_and(lax.rem(t + 1, CPS) == 0,
                                   t + 1 < NCHUNK)

        @pl.when(t >= 2)
        def _():
            pltpu.make_async_copy(
                bufB, acc_sh.at[dst_v.at[sl, r]], ssB).wait()

        @pl.when(boundary)
        def _():
            pltpu.make_async_copy(h_hbm.at[src_v.at[r]], bufA, sgA).wait()
            stage((t + 1) // CPS)

        @pl.when(t + 1 < NCHUNK)
        def _():
            pltpu.async_copy(
                h_hbm.at[src_v.at[lax.rem(t + 1, CPS)]], bufB, sgB)

        @pl.when(jnp.logical_not(boundary))
        def _():
            pltpu.make_async_copy(h_hbm.at[src_v.at[r]], bufA, sgA).wait()

        scale(bufA, r)
        pltpu.async_copy(bufA, acc_sh.at[dst_v.at[sl, r]], ssA, add=True)

        @pl.when(boundary)
        def _():
            stage_val((t + 1) // CPS)

    stage(0)
    stage_val(0)

    def zdrain(k, carry):
        pltpu.make_async_copy(rows_v, acc_sh.at[pl.ds(base + k * K, K)],
                              sg1).wait()
        return carry

    lax.fori_loop(0, nfull, zdrain, 0)

    @pl.when(s < 15)
    def _():
        pltpu.make_async_copy(rows_v.at[pl.ds(0, rem)],
                              acc_sh.at[pl.ds(base + nfull * K, rem)],
                              sg1).wait()

    @pl.when(s == 15)
    def _():
        def ztailw(k, carry):
            pltpu.make_async_copy(
                rows_v, acc_sh.at[pl.ds(base + nfull * K + k * K, K)],
                sg1).wait()
            return carry
        lax.fori_loop(0, ntail, ztailw, 0)

    plsc.subcore_barrier()

    pltpu.async_copy(h_hbm.at[src_v.at[0]], rows0_v, sg0)

    def triple(p, carry):
        t = 3 * p
        step(t, rows0_v, sg0, ss0, rows1_v, sg1, ss1)
        step(t + 1, rows1_v, sg1, ss1, rows2_v, sg2, ss2)
        step(t + 2, rows2_v, sg2, ss2, rows0_v, sg0, ss0)
        return carry

    ntr = (NCHUNK - 2) // 3
    lax.fori_loop(0, ntr, triple, 0)
    step(NCHUNK - 2, rows0_v, sg0, ss0, rows1_v, sg1, ss1)
    step(NCHUNK - 1, rows1_v, sg1, ss1, rows2_v, sg2, ss2)
    lastsl = lax.rem((NCHUNK - 1) // CPS, 2)
    pltpu.make_async_copy(rows0_v, acc_sh.at[dst_v.at[lastsl, 0]], ss0).wait()
    pltpu.make_async_copy(rows1_v, acc_sh.at[dst_v.at[lastsl, 0]], ss1).wait()

    plsc.subcore_barrier()

    @pl.when(s < 15)
    def _():
        pltpu.sync_copy(acc_sh.at[pl.ds(base, ROWS_A)],
                        out_hbm.at[c, pl.ds(base, ROWS_A)])

    @pl.when(s == 15)
    def _():
        pltpu.sync_copy(acc_sh.at[pl.ds(base, ROWS_LAST)],
                        out_hbm.at[c, pl.ds(base, ROWS_LAST)])


_sc_kernel = functools.partial(
    pl.kernel,
    out_type=jax.ShapeDtypeStruct((NC, N, D), jnp.float32),
    mesh=plsc.VectorSubcoreMesh(
        core_axis_name="c", subcore_axis_name="s",
        num_cores=NC, num_subcores=NS),
    scratch_types=[
        pltpu.VMEM((CPS, K), jnp.int32),
        pltpu.VMEM((2, CPS, K), jnp.int32),
        pltpu.VMEM((CPS, K), jnp.float32),
        pltpu.VMEM((K, D), jnp.float32),
        pltpu.VMEM((K, D), jnp.float32),
        pltpu.VMEM((K, D), jnp.float32),
        pltpu.VMEM_SHARED((N, D), jnp.float32),
        pltpu.SemaphoreType.DMA,
        pltpu.SemaphoreType.DMA,
        pltpu.SemaphoreType.DMA,
        pltpu.SemaphoreType.DMA,
        pltpu.SemaphoreType.DMA,
        pltpu.SemaphoreType.DMA,
    ],
)(_sc_body)


@jax.jit
def kernel(adj_edge_index, adj_values, features, W, b):
    src = adj_edge_index[0].astype(jnp.int32).reshape(NC, NS, SB, CPS, K)
    dst = adj_edge_index[1].astype(jnp.int32).reshape(NC, NS, SB, CPS, K)
    val = adj_values.reshape(NC, NS, SB, CPS, K)
    h = _linear(features, W.T, b.reshape(1, D))
    out2 = _sc_kernel(src, dst, val, h)
    return _combine(out2)


# speedup vs baseline: 8.7845x; 1.0297x over previous
import functools

import jax
import jax.numpy as jnp
from jax import lax
from jax.experimental import pallas as pl
from jax.experimental.pallas import tpu as pltpu
from jax.experimental.pallas import tpu_sc as plsc

N = 10000
E = 320000
D = 128
NC = 2
NS = 16
EPW = E // (NC * NS)
K = 80
CPS = 25
SB = EPW // (K * CPS)
NCHUNK = SB * CPS
ROWS_A = 624
ROWS_LAST = N - 15 * ROWS_A
RB = N // 5


def _linear_body(f_ref, w_ref, b_ref, o_ref):
    o_ref[...] = (
        lax.dot_general(f_ref[...], w_ref[...], (((1,), (1,)), ((), ())),
                        preferred_element_type=jnp.float32)
        + b_ref[...]
    )


def _linear(features, W, b2):
    return pl.pallas_call(
        _linear_body,
        grid=(5,),
        in_specs=[
            pl.BlockSpec((RB, D), lambda i: (i, 0)),
            pl.BlockSpec((D, D), lambda i: (0, 0)),
            pl.BlockSpec((1, D), lambda i: (0, 0)),
        ],
        out_specs=pl.BlockSpec((RB, D), lambda i: (i, 0)),
        out_shape=jax.ShapeDtypeStruct((N, D), jnp.float32),
    )(features, W, b2)


def _combine_body(a_ref, b_ref, o_ref):
    o_ref[...] = a_ref[0] + b_ref[0]


def _combine(out2):
    return pl.pallas_call(
        _combine_body,
        grid=(5,),
        in_specs=[
            pl.BlockSpec((1, RB, D), lambda i: (0, i, 0)),
            pl.BlockSpec((1, RB, D), lambda i: (1, i, 0)),
        ],
        out_specs=pl.BlockSpec((RB, D), lambda i: (i, 0)),
        out_shape=jax.ShapeDtypeStruct((N, D), jnp.float32),
    )(out2, out2)


def _sc_body(src_hbm, dst_hbm, val_hbm, h_hbm, out_hbm,
             src_v, dst_v, val_v, rows0_v, rows1_v, rows2_v, acc_sh,
             sg0, sg1, sg2, ss0, ss1, ss2):
    c = lax.axis_index("c")
    s = lax.axis_index("s")
    rows_v = rows0_v

    zero16 = jnp.zeros((16,), jnp.float32)

    def zrow(r, carry):
        for g in range(D // 16):
            rows_v[r, pl.ds(g * 16, 16)] = zero16
        return carry

    lax.fori_loop(0, K, zrow, 0)

    base = s * ROWS_A
    nfull = ROWS_A // K

    rem = ROWS_A - nfull * K
    ntail = (ROWS_LAST - nfull * K) // K

    def zacc(k, carry):
        pltpu.async_copy(rows_v, acc_sh.at[pl.ds(base + k * K, K)], sg1)
        return carry

    lax.fori_loop(0, nfull, zacc, 0)

    @pl.when(s < 15)
    def _():
        pltpu.async_copy(rows_v.at[pl.ds(0, rem)],
                         acc_sh.at[pl.ds(base + nfull * K, rem)], sg1)

    @pl.when(s == 15)
    def _():
        def ztail(k, carry):
            pltpu.async_copy(rows_v,
                             acc_sh.at[pl.ds(base + nfull * K + k * K, K)],
                             sg1)
            return carry
        lax.fori_loop(0, ntail, ztail, 0)

    def stage(u):
        pltpu.sync_copy(src_hbm.at[c, s, u], src_v)
        pltpu.sync_copy(dst_hbm.at[c, s, u], dst_v.at[lax.rem(u, 2)])

    def stage_val(u):
        pltpu.sync_copy(val_hbm.at[c, s, u], val_v)

    def scale(buf, r):
        for g in range(K // 16):
            vv = val_v[r, pl.ds(g * 16, 16)]
            for e in range(16):
                sp = vv.at[jnp.full((16,), e, jnp.int32)].get(
                    mode="promise_in_bounds")
                row = g * 16 + e
                for q in range(D // 16):
                    slc = pl.ds(q * 16, 16)
                    buf[row, slc] = buf[row, slc] * sp

    def step(t, bufA, sgA, ssA, bufB, sgB, ssB):
        t = jnp.int32(t)
        sl = lax.rem(t // CPS, 2)
        r = lax.rem(t, CPS)

        boundary = jnp.logical_and(lax.rem(t + 1, CPS) == 0,
                                   t + 1 < NCHUNK)

        @pl.when(t >= 2)
        def _():
            pltpu.make_async_copy(
                bufB, acc_sh.at[dst_v.at[sl, r]], ssB).wait()

        @pl.when(boundary)
        def _():
            pltpu.make_async_copy(h_hbm.at[src_v.at[r]], bufA, sgA).wait()
            stage((t + 1) // CPS)

        @pl.when(t + 1 < NCHUNK)
        def _():
            pltpu.async_copy(
                h_hbm.at[src_v.at[lax.rem(t + 1, CPS)]], bufB, sgB)

        @pl.when(jnp.logical_not(boundary))
        def _():
            pltpu.make_async_copy(h_hbm.at[src_v.at[r]], bufA, sgA).wait()

        scale(bufA, r)
        pltpu.async_copy(bufA, acc_sh.at[dst_v.at[sl, r]], ssA, add=True)

        @pl.when(boundary)
        def _():
            stage_val((t + 1) // CPS)

    stage(0)
    stage_val(0)

    def zdrain(k, carry):
        pltpu.make_async_copy(rows_v, acc_sh.at[pl.ds(base + k * K, K)],
                              sg1).wait()
        return carry

    lax.fori_loop(0, nfull, zdrain, 0)

    @pl.when(s < 15)
    def _():
        pltpu.make_async_copy(rows_v.at[pl.ds(0, rem)],
                              acc_sh.at[pl.ds(base + nfull * K, rem)],
                              sg1).wait()

    @pl.when(s == 15)
    def _():
        def ztailw(k, carry):
            pltpu.make_async_copy(
                rows_v, acc_sh.at[pl.ds(base + nfull * K + k * K, K)],
                sg1).wait()
            return carry
        lax.fori_loop(0, ntail, ztailw, 0)

    plsc.subcore_barrier()

    pltpu.async_copy(h_hbm.at[src_v.at[0]], rows0_v, sg0)

    def triple(p, carry):
        t = 3 * p
        step(t, rows0_v, sg0, ss0, rows1_v, sg1, ss1)
        step(t + 1, rows1_v, sg1, ss1, rows2_v, sg2, ss2)
        step(t + 2, rows2_v, sg2, ss2, rows0_v, sg0, ss0)
        return carry

    ntr = (NCHUNK - 2) // 3
    lax.fori_loop(0, ntr, triple, 0)
    step(NCHUNK - 2, rows0_v, sg0, ss0, rows1_v, sg1, ss1)
    step(NCHUNK - 1, rows1_v, sg1, ss1, rows2_v, sg2, ss2)
    lastsl = lax.rem((NCHUNK - 1) // CPS, 2)
    pltpu.make_async_copy(rows0_v, acc_sh.at[dst_v.at[lastsl, 0]], ss0).wait()
    pltpu.make_async_copy(rows1_v, acc_sh.at[dst_v.at[lastsl, 0]], ss1).wait()

    plsc.subcore_barrier()

    @pl.when(s < 15)
    def _():
        pltpu.sync_copy(acc_sh.at[pl.ds(base, ROWS_A)],
                        out_hbm.at[c, pl.ds(base, ROWS_A)])

    @pl.when(s == 15)
    def _():
        pltpu.sync_copy(acc_sh.at[pl.ds(base, ROWS_LAST)],
                        out_hbm.at[c, pl.ds(base, ROWS_LAST)])


_sc_kernel = functools.partial(
    pl.kernel,
    out_type=jax.ShapeDtypeStruct((NC, N, D), jnp.float32),
    mesh=plsc.VectorSubcoreMesh(
        core_axis_name="c", subcore_axis_name="s",
        num_cores=NC, num_subcores=NS),
    scratch_types=[
        pltpu.VMEM((CPS, K), jnp.int32),
        pltpu.VMEM((2, CPS, K), jnp.int32),
        pltpu.VMEM((CPS, K), jnp.float32),
        pltpu.VMEM((K, D), jnp.float32),
        pltpu.VMEM((K, D), jnp.float32),
        pltpu.VMEM((K, D), jnp.float32),
        pltpu.VMEM_SHARED((N, D), jnp.float32),
        pltpu.SemaphoreType.DMA,
        pltpu.SemaphoreType.DMA,
        pltpu.SemaphoreType.DMA,
        pltpu.SemaphoreType.DMA,
        pltpu.SemaphoreType.DMA,
        pltpu.SemaphoreType.DMA,
    ],
)(_sc_body)


@jax.jit
def kernel(adj_edge_index, adj_values, features, W, b):
    src = adj_edge_index[0].astype(jnp.int32).reshape(NC, NS, SB, CPS, K)
    dst = adj_edge_index[1].astype(jnp.int32).reshape(NC, NS, SB, CPS, K)
    val = adj_values.reshape(NC, NS, SB, CPS, K)
    h = _linear(features, W, b.reshape(1, D))
    out2 = _sc_kernel(src, dst, val, h)
    return _combine(out2)
